# baseline (speedup 1.0000x reference)
import functools

import jax
import jax.numpy as jnp
from jax import lax
from jax.experimental import pallas as pl
from jax.experimental.pallas import tpu as pltpu
from jax.experimental.pallas import tpu_sc as plsc

N = 10000
E = 320000
D = 128
B = 64
STEPS = 3

NC = 2
NS = 16
NW = NC * NS
CHUNK = 128
EROWS = 2560
RPW = EROWS // NW
HALF = RPW // 2
EPAD = EROWS * CHUNK - E
NP = 10112
RPS = NP // NS

_MESH = plsc.VectorSubcoreMesh(
    core_axis_name="c", subcore_axis_name="s", num_cores=NC, num_subcores=NS)


def _deg_body(dst_hbm, ones_hbm, zfeat_hbm, out_hbm, acc, ones_vm, idx_vm):
    c = lax.axis_index("c")
    s = lax.axis_index("s")
    base = (c * NS + s) * RPW

    pltpu.sync_copy(dst_hbm.at[pl.ds(base, RPW)], idx_vm)
    pltpu.sync_copy(ones_hbm, ones_vm)
    pltpu.sync_copy(zfeat_hbm, acc.at[pl.ds(s * RPS, RPS)])
    plsc.subcore_barrier()

    @pl.loop(0, RPW)
    def _scatter(g):
        pltpu.sync_copy(ones_vm, acc.at[idx_vm.at[g]], add=True)

    plsc.subcore_barrier()
    pltpu.sync_copy(acc.at[pl.ds(s * RPS, RPS)],
                    out_hbm.at[c].at[pl.ds(s * RPS, RPS)])


@functools.partial(
    pl.kernel,
    out_type=jax.ShapeDtypeStruct((NC, NP, D), jnp.float32),
    mesh=_MESH,
    scratch_types=[
        pltpu.VMEM_SHARED((NP, D), jnp.float32),
        pltpu.VMEM((CHUNK, D), jnp.float32),
        pltpu.VMEM((RPW, CHUNK), jnp.int32),
    ],
)
def _deg_sc(dst_hbm, ones_hbm, zfeat_hbm, out_hbm, acc, ones_vm, idx_vm):
    _deg_body(dst_hbm, ones_hbm, zfeat_hbm, out_hbm, acc, ones_vm, idx_vm)


def _agg_body(h_hbm, src_hbm, dst_hbm, zfeat_hbm, out_hbm, acc,
              rows0, rows1, idxs_vm, idxd_vm, sem0, sem1):
    c = lax.axis_index("c")
    s = lax.axis_index("s")
    base = (c * NS + s) * RPW

    rows = (rows0, rows1)
    sems = (sem0, sem1)

    def fire(g, b):
        pltpu.async_copy(h_hbm.at[idxs_vm.at[g]], rows[b], sems[b])

    def drain_scatter(g, b):
        pltpu.make_async_copy(h_hbm.at[idxs_vm.at[g]], rows[b], sems[b]).wait()
        pltpu.sync_copy(rows[b], acc.at[idxd_vm.at[g]], add=True)

    def phase(off):
        pltpu.sync_copy(src_hbm.at[pl.ds(base + off, HALF)], idxs_vm)
        pltpu.sync_copy(dst_hbm.at[pl.ds(base + off, HALF)], idxd_vm)
        fire(0, 0)

        @pl.loop(0, (HALF - 2) // 2)
        def _pipeline(i):
            g = 2 * i
            fire(g + 1, 1)
            drain_scatter(g, 0)
            fire(g + 2, 0)
            drain_scatter(g + 1, 1)

        fire(HALF - 1, 1)
        drain_scatter(HALF - 2, 0)
        drain_scatter(HALF - 1, 1)

    pltpu.sync_copy(zfeat_hbm, acc.at[pl.ds(s * RPS, RPS)])
    plsc.subcore_barrier()

    phase(0)
    phase(HALF)

    plsc.subcore_barrier()
    pltpu.sync_copy(acc.at[pl.ds(s * RPS, RPS)],
                    out_hbm.at[c].at[pl.ds(s * RPS, RPS)])


@functools.partial(
    pl.kernel,
    out_type=jax.ShapeDtypeStruct((NC, NP, D), jnp.float32),
    mesh=_MESH,
    scratch_types=[
        pltpu.VMEM_SHARED((NP, D), jnp.float32),
        pltpu.VMEM((CHUNK, D), jnp.float32),
        pltpu.VMEM((CHUNK, D), jnp.float32),
        pltpu.VMEM((HALF, CHUNK), jnp.int32),
        pltpu.VMEM((HALF, CHUNK), jnp.int32),
        pltpu.SemaphoreType.DMA,
        pltpu.SemaphoreType.DMA,
    ],
)
def _agg_sc(h_hbm, src_hbm, dst_hbm, zfeat_hbm, out_hbm, acc,
            rows0, rows1, idxs_vm, idxd_vm, sem0, sem1):
    _agg_body(h_hbm, src_hbm, dst_hbm, zfeat_hbm, out_hbm, acc,
              rows0, rows1, idxs_vm, idxd_vm, sem0, sem1)


def _dinv_from(degp):
    deg = degp[0, :N, 0:1] + degp[1, :N, 0:1] + 1.0
    return lax.rsqrt(deg)


def _k1_body(x_ref, w_ref, degp_ref, o_ref):
    dinv = _dinv_from(degp_ref[...])
    h = jnp.dot(x_ref[...], w_ref[...], preferred_element_type=jnp.float32)
    o_ref[0:N] = h * dinv
    o_ref[N:NP] = jnp.zeros((NP - N, D), jnp.float32)


def _k1(x, W1, degp):
    return pl.pallas_call(
        _k1_body,
        out_shape=jax.ShapeDtypeStruct((NP, D), jnp.float32),
    )(x, W1, degp)


def _k2_body(p_ref, hs1_ref, degp_ref, b1_ref, w2_ref, o_ref):
    dinv = _dinv_from(degp_ref[...])
    tot = p_ref[0, :N] + p_ref[1, :N] + hs1_ref[0:N]
    h1 = jnp.maximum(tot * dinv + b1_ref[...], 0.0)
    h2 = jnp.dot(h1, w2_ref[...], preferred_element_type=jnp.float32)
    o_ref[0:N] = h2 * dinv
    o_ref[N:NP] = jnp.zeros((NP - N, D), jnp.float32)


def _k2(p, hs1, degp, b1, W2):
    return pl.pallas_call(
        _k2_body,
        out_shape=jax.ShapeDtypeStruct((NP, D), jnp.float32),
    )(p, hs1, degp, b1.reshape(1, D), W2)


def _k3_body(p_ref, hs2_ref, degp_ref, b2_ref, batch_ref, wih_ref, whh_ref,
             bih_ref, bhh_ref, o_ref):
    dinv = _dinv_from(degp_ref[...])
    h2 = (p_ref[0, :N] + p_ref[1, :N] + hs2_ref[0:N]) * dinv + b2_ref[...]

    seg = batch_ref[0:1, :]
    bids = lax.broadcasted_iota(jnp.int32, (B, N), 0)
    member = bids == seg

    q_star = jnp.zeros((B, 2 * D), jnp.float32)
    h = jnp.zeros((B, D), jnp.float32)
    c = jnp.zeros((B, D), jnp.float32)
    wih = wih_ref[...]
    whh = whh_ref[...]
    bias = bih_ref[...] + bhh_ref[...]

    for _ in range(STEPS):
        gates = (
            lax.dot_general(q_star, wih, (((1,), (1,)), ((), ())),
                            preferred_element_type=jnp.float32)
            + lax.dot_general(h, whh, (((1,), (1,)), ((), ())),
                              preferred_element_type=jnp.float32)
            + bias)
        ig = jax.nn.sigmoid(gates[:, 0:D])
        fg = jax.nn.sigmoid(gates[:, D:2 * D])
        gg = jnp.tanh(gates[:, 2 * D:3 * D])
        og = jax.nn.sigmoid(gates[:, 3 * D:4 * D])
        c = fg * c + ig * gg
        h = og * jnp.tanh(c)

        e = lax.dot_general(h, h2, (((1,), (1,)), ((), ())),
                            preferred_element_type=jnp.float32)
        em = jnp.where(member, e, -jnp.inf)
        emax = jnp.max(em, axis=1, keepdims=True)
        emax = jnp.where(emax < -3e38, 0.0, emax)
        a = jnp.exp(em - emax)
        ssum = jnp.sum(a, axis=1, keepdims=True)
        r = lax.dot_general(a, h2, (((1,), (0,)), ((), ())),
                            preferred_element_type=jnp.float32)
        r = r / (ssum + 1e-16)
        q_star = jnp.concatenate([h, r], axis=1)

    o_ref[...] = q_star


def _k3(p2, hs2, degp, b2, batch, w_ih, w_hh, b_ih, b_hh):
    batch8 = jnp.broadcast_to(batch[None, :], (8, N))
    return pl.pallas_call(
        _k3_body,
        out_shape=jax.ShapeDtypeStruct((B, 2 * D), jnp.float32),
    )(p2, hs2, degp, b2.reshape(1, D), batch8, w_ih, w_hh,
      b_ih.reshape(1, 4 * D), b_hh.reshape(1, 4 * D))


def kernel(x, edge_index, batch, W1, b1, W2, b2, w_ih, w_hh, b_ih, b_hh):
    pad = jnp.full((EPAD,), N, dtype=edge_index.dtype)
    src = jnp.concatenate([edge_index[0], pad]).reshape(EROWS, CHUNK)
    dst = jnp.concatenate([edge_index[1], pad]).reshape(EROWS, CHUNK)

    onehot = jnp.zeros((CHUNK, D), jnp.float32).at[:, 0].set(1.0)
    zfeat = jnp.zeros((RPS, D), jnp.float32)

    degp = _deg_sc(dst, onehot, zfeat)
    hs1 = _k1(x, W1, degp)
    p1 = _agg_sc(hs1, src, dst, zfeat)
    hs2 = _k2(p1, hs1, degp, b1, W2)
    p2 = _agg_sc(hs2, src, dst, zfeat)
    return _k3(p2, hs2, degp, b2, batch, w_ih, w_hh, b_ih, b_hh)


# speedup vs baseline: 3.0896x; 3.0896x over previous
import functools

import jax
import jax.numpy as jnp
from jax import lax
from jax.experimental import pallas as pl
from jax.experimental.pallas import tpu as pltpu
from jax.experimental.pallas import tpu_sc as plsc

N = 10000
E = 320000
D = 128
B = 64
STEPS = 3

NC = 2
NS = 16
NW = NC * NS
CHUNK = 128
EROWS = 2560
RPW = EROWS // NW
HALF = RPW // 2
EPAD = EROWS * CHUNK - E
NP = 10112
RPS = NP // NS

_MESH = plsc.VectorSubcoreMesh(
    core_axis_name="c", subcore_axis_name="s", num_cores=NC, num_subcores=NS)


def _deg_body(dst_hbm, ones_hbm, zfeat_hbm, out_hbm, acc, ones_vm, idx_vm):
    c = lax.axis_index("c")
    s = lax.axis_index("s")
    base = (c * NS + s) * RPW

    pltpu.sync_copy(dst_hbm.at[pl.ds(base, RPW)], idx_vm)
    pltpu.sync_copy(ones_hbm, ones_vm)
    pltpu.sync_copy(zfeat_hbm, acc.at[pl.ds(s * RPS, RPS)])
    plsc.subcore_barrier()

    @pl.loop(0, RPW)
    def _scatter(g):
        pltpu.sync_copy(ones_vm, acc.at[idx_vm.at[g]], add=True)

    plsc.subcore_barrier()
    pltpu.sync_copy(acc.at[pl.ds(s * RPS, RPS)],
                    out_hbm.at[c].at[pl.ds(s * RPS, RPS)])


@functools.partial(
    pl.kernel,
    out_type=jax.ShapeDtypeStruct((NC, NP, D), jnp.float32),
    mesh=_MESH,
    scratch_types=[
        pltpu.VMEM_SHARED((NP, D), jnp.float32),
        pltpu.VMEM((CHUNK, D), jnp.float32),
        pltpu.VMEM((RPW, CHUNK), jnp.int32),
    ],
)
def _deg_sc(dst_hbm, ones_hbm, zfeat_hbm, out_hbm, acc, ones_vm, idx_vm):
    _deg_body(dst_hbm, ones_hbm, zfeat_hbm, out_hbm, acc, ones_vm, idx_vm)


def _agg_body(h_hbm, src_hbm, dst_hbm, zfeat_hbm, out_hbm, acc,
              rows0, rows1, idxs_vm, idxd_vm, sem0, sem1):
    c = lax.axis_index("c")
    s = lax.axis_index("s")
    base = (c * NS + s) * RPW

    rows = (rows0, rows1)
    sems = (sem0, sem1)

    def fire(g, b):
        pltpu.async_copy(h_hbm.at[idxs_vm.at[g]], rows[b], sems[b])

    def drain_scatter(g, b):
        pltpu.make_async_copy(h_hbm.at[idxs_vm.at[g]], rows[b], sems[b]).wait()
        pltpu.sync_copy(rows[b], acc.at[idxd_vm.at[g]], add=True)

    def phase(off):
        pltpu.sync_copy(src_hbm.at[pl.ds(base + off, HALF)], idxs_vm)
        pltpu.sync_copy(dst_hbm.at[pl.ds(base + off, HALF)], idxd_vm)
        fire(0, 0)

        @pl.loop(0, (HALF - 2) // 2)
        def _pipeline(i):
            g = 2 * i
            fire(g + 1, 1)
            drain_scatter(g, 0)
            fire(g + 2, 0)
            drain_scatter(g + 1, 1)

        fire(HALF - 1, 1)
        drain_scatter(HALF - 2, 0)
        drain_scatter(HALF - 1, 1)

    pltpu.sync_copy(zfeat_hbm, acc.at[pl.ds(s * RPS, RPS)])
    plsc.subcore_barrier()

    phase(0)
    phase(HALF)

    plsc.subcore_barrier()
    pltpu.sync_copy(acc.at[pl.ds(s * RPS, RPS)],
                    out_hbm.at[c].at[pl.ds(s * RPS, RPS)])


@functools.partial(
    pl.kernel,
    out_type=jax.ShapeDtypeStruct((NC, NP, D), jnp.float32),
    mesh=_MESH,
    scratch_types=[
        pltpu.VMEM_SHARED((NP, D), jnp.float32),
        pltpu.VMEM((CHUNK, D), jnp.float32),
        pltpu.VMEM((CHUNK, D), jnp.float32),
        pltpu.VMEM((HALF, CHUNK), jnp.int32),
        pltpu.VMEM((HALF, CHUNK), jnp.int32),
        pltpu.SemaphoreType.DMA,
        pltpu.SemaphoreType.DMA,
    ],
)
def _agg_sc(h_hbm, src_hbm, dst_hbm, zfeat_hbm, out_hbm, acc,
            rows0, rows1, idxs_vm, idxd_vm, sem0, sem1):
    _agg_body(h_hbm, src_hbm, dst_hbm, zfeat_hbm, out_hbm, acc,
              rows0, rows1, idxs_vm, idxd_vm, sem0, sem1)


def _dinv_from(degp):
    deg = degp[0, :N, 0:1] + degp[1, :N, 0:1] + 1.0
    return lax.rsqrt(deg)


def _k1_body(x_ref, w_ref, degp_ref, o_ref):
    dinv = _dinv_from(degp_ref[...])
    h = jnp.dot(x_ref[...], w_ref[...], preferred_element_type=jnp.float32)
    o_ref[0:N] = h * dinv
    o_ref[N:NP] = jnp.zeros((NP - N, D), jnp.float32)


def _k1(x, W1, degp):
    return pl.pallas_call(
        _k1_body,
        out_shape=jax.ShapeDtypeStruct((NP, D), jnp.float32),
    )(x, W1, degp)


def _k2_body(p_ref, hs1_ref, degp_ref, b1_ref, w2_ref, o_ref):
    dinv = _dinv_from(degp_ref[...])
    tot = p_ref[0, :N] + p_ref[1, :N] + hs1_ref[0:N]
    h1 = jnp.maximum(tot * dinv + b1_ref[...], 0.0)
    h2 = jnp.dot(h1, w2_ref[...], preferred_element_type=jnp.float32)
    o_ref[0:N] = h2 * dinv
    o_ref[N:NP] = jnp.zeros((NP - N, D), jnp.float32)


def _k2(p, hs1, degp, b1, W2):
    return pl.pallas_call(
        _k2_body,
        out_shape=jax.ShapeDtypeStruct((NP, D), jnp.float32),
    )(p, hs1, degp, b1.reshape(1, D), W2)


def _k3_body(p_ref, hs2_ref, degp_ref, b2_ref, batch_ref, wih_ref, whh_ref,
             bih_ref, bhh_ref, o_ref):
    dinv = _dinv_from(degp_ref[...])
    h2 = (p_ref[0, :N] + p_ref[1, :N] + hs2_ref[0:N]) * dinv + b2_ref[...]

    seg = batch_ref[0:1, :]
    bids = lax.broadcasted_iota(jnp.int32, (B, N), 0)
    member = bids == seg

    q_star = jnp.zeros((B, 2 * D), jnp.float32)
    h = jnp.zeros((B, D), jnp.float32)
    c = jnp.zeros((B, D), jnp.float32)
    wih = wih_ref[...]
    whh = whh_ref[...]
    bias = bih_ref[...] + bhh_ref[...]

    for _ in range(STEPS):
        gates = (
            lax.dot_general(q_star, wih, (((1,), (1,)), ((), ())),
                            preferred_element_type=jnp.float32)
            + lax.dot_general(h, whh, (((1,), (1,)), ((), ())),
                              preferred_element_type=jnp.float32)
            + bias)
        ig = jax.nn.sigmoid(gates[:, 0:D])
        fg = jax.nn.sigmoid(gates[:, D:2 * D])
        gg = jnp.tanh(gates[:, 2 * D:3 * D])
        og = jax.nn.sigmoid(gates[:, 3 * D:4 * D])
        c = fg * c + ig * gg
        h = og * jnp.tanh(c)

        e = lax.dot_general(h, h2, (((1,), (1,)), ((), ())),
                            preferred_element_type=jnp.float32)
        em = jnp.where(member, e, -jnp.inf)
        emax = jnp.max(em, axis=1, keepdims=True)
        emax = jnp.where(emax < -3e38, 0.0, emax)
        a = jnp.exp(em - emax)
        ssum = jnp.sum(a, axis=1, keepdims=True)
        r = lax.dot_general(a, h2, (((1,), (0,)), ((), ())),
                            preferred_element_type=jnp.float32)
        r = r / (ssum + 1e-16)
        q_star = jnp.concatenate([h, r], axis=1)

    o_ref[...] = q_star


def _k3(p2, hs2, degp, b2, batch, w_ih, w_hh, b_ih, b_hh):
    batch8 = jnp.broadcast_to(batch[None, :], (8, N))
    return pl.pallas_call(
        _k3_body,
        out_shape=jax.ShapeDtypeStruct((B, 2 * D), jnp.float32),
    )(p2, hs2, degp, b2.reshape(1, D), batch8, w_ih, w_hh,
      b_ih.reshape(1, 4 * D), b_hh.reshape(1, 4 * D))


def kernel(x, edge_index, batch, W1, b1, W2, b2, w_ih, w_hh, b_ih, b_hh):
    pad = (N + jnp.arange(EPAD, dtype=edge_index.dtype) % (NP - N))
    src = jnp.concatenate([edge_index[0], pad]).reshape(EROWS, CHUNK)
    dst = jnp.concatenate([edge_index[1], pad]).reshape(EROWS, CHUNK)

    onehot = jnp.zeros((CHUNK, D), jnp.float32).at[:, 0].set(1.0)
    zfeat = jnp.zeros((RPS, D), jnp.float32)

    degp = _deg_sc(dst, onehot, zfeat)
    hs1 = _k1(x, W1, degp)
    p1 = _agg_sc(hs1, src, dst, zfeat)
    hs2 = _k2(p1, hs1, degp, b1, W2)
    p2 = _agg_sc(hs2, src, dst, zfeat)
    return _k3(p2, hs2, degp, b2, batch, w_ih, w_hh, b_ih, b_hh)


# speedup vs baseline: 3.5665x; 1.1544x over previous
import functools

import jax
import jax.numpy as jnp
from jax import lax
from jax.experimental import pallas as pl
from jax.experimental.pallas import tpu as pltpu
from jax.experimental.pallas import tpu_sc as plsc

N = 10000
E = 320000
D = 128
B = 64
STEPS = 3

NC = 2
NS = 16
NW = NC * NS
CHUNK = 128
EROWS = 2560
RPW = EROWS // NW
HALF = RPW // 2
EPAD = EROWS * CHUNK - E
NP = 10112
RPS = NP // NS

_MESH = plsc.VectorSubcoreMesh(
    core_axis_name="c", subcore_axis_name="s", num_cores=NC, num_subcores=NS)


def _deg_body(dst_hbm, ones_hbm, zrow_hbm, out_hbm, acc, ones_vm, idx_vm):
    c = lax.axis_index("c")
    s = lax.axis_index("s")
    base = (c * NS + s) * RPW

    pltpu.sync_copy(dst_hbm.at[pl.ds(base, RPW)], idx_vm)
    pltpu.sync_copy(ones_hbm, ones_vm)
    pltpu.sync_copy(zrow_hbm, acc.at[pl.ds(s * RPS, RPS)])
    plsc.subcore_barrier()

    @pl.loop(0, RPW)
    def _scatter(g):
        pltpu.sync_copy(ones_vm, acc.at[idx_vm.at[g]], add=True)

    plsc.subcore_barrier()
    pltpu.sync_copy(acc.at[pl.ds(s * RPS, RPS)],
                    out_hbm.at[c].at[pl.ds(s * RPS, RPS)])


@functools.partial(
    pl.kernel,
    out_type=jax.ShapeDtypeStruct((NC, NP, 16), jnp.float32),
    mesh=_MESH,
    scratch_types=[
        pltpu.VMEM_SHARED((NP, 16), jnp.float32),
        pltpu.VMEM((CHUNK, 16), jnp.float32),
        pltpu.VMEM((RPW, CHUNK), jnp.int32),
    ],
    compiler_params=pltpu.CompilerParams(use_tc_tiling_on_sc=False),
)
def _deg_sc(dst_hbm, ones_hbm, zrow_hbm, out_hbm, acc, ones_vm, idx_vm):
    _deg_body(dst_hbm, ones_hbm, zrow_hbm, out_hbm, acc, ones_vm, idx_vm)


def _agg_body(h_hbm, src_hbm, dst_hbm, zfeat_hbm, out_hbm, acc,
              rows0, rows1, idxs_vm, idxd_vm, sem0, sem1):
    c = lax.axis_index("c")
    s = lax.axis_index("s")
    base = (c * NS + s) * RPW

    rows = (rows0, rows1)
    sems = (sem0, sem1)

    def fire(g, b):
        pltpu.async_copy(h_hbm.at[idxs_vm.at[g]], rows[b], sems[b])

    def drain_scatter(g, b):
        pltpu.make_async_copy(h_hbm.at[idxs_vm.at[g]], rows[b], sems[b]).wait()
        pltpu.sync_copy(rows[b], acc.at[idxd_vm.at[g]], add=True)

    def phase(off):
        pltpu.sync_copy(src_hbm.at[pl.ds(base + off, HALF)], idxs_vm)
        pltpu.sync_copy(dst_hbm.at[pl.ds(base + off, HALF)], idxd_vm)
        fire(0, 0)

        @pl.loop(0, (HALF - 2) // 2)
        def _pipeline(i):
            g = 2 * i
            fire(g + 1, 1)
            drain_scatter(g, 0)
            fire(g + 2, 0)
            drain_scatter(g + 1, 1)

        fire(HALF - 1, 1)
        drain_scatter(HALF - 2, 0)
        drain_scatter(HALF - 1, 1)

    pltpu.sync_copy(zfeat_hbm, acc.at[pl.ds(s * RPS, RPS)])
    plsc.subcore_barrier()

    phase(0)
    phase(HALF)

    plsc.subcore_barrier()
    pltpu.sync_copy(acc.at[pl.ds(s * RPS, RPS)],
                    out_hbm.at[c].at[pl.ds(s * RPS, RPS)])


@functools.partial(
    pl.kernel,
    out_type=jax.ShapeDtypeStruct((NC, NP, D), jnp.float32),
    mesh=_MESH,
    scratch_types=[
        pltpu.VMEM_SHARED((NP, D), jnp.float32),
        pltpu.VMEM((CHUNK, D), jnp.float32),
        pltpu.VMEM((CHUNK, D), jnp.float32),
        pltpu.VMEM((HALF, CHUNK), jnp.int32),
        pltpu.VMEM((HALF, CHUNK), jnp.int32),
        pltpu.SemaphoreType.DMA,
        pltpu.SemaphoreType.DMA,
    ],
)
def _agg_sc(h_hbm, src_hbm, dst_hbm, zfeat_hbm, out_hbm, acc,
            rows0, rows1, idxs_vm, idxd_vm, sem0, sem1):
    _agg_body(h_hbm, src_hbm, dst_hbm, zfeat_hbm, out_hbm, acc,
              rows0, rows1, idxs_vm, idxd_vm, sem0, sem1)


def _dinv_from(degp):
    deg = degp[0, :N, 0:1] + degp[1, :N, 0:1] + 1.0
    return lax.rsqrt(deg)


def _k1_body(x_ref, w_ref, degp_ref, o_ref):
    dinv = _dinv_from(degp_ref[...])
    h = jnp.dot(x_ref[...], w_ref[...], preferred_element_type=jnp.float32)
    o_ref[0:N] = h * dinv
    o_ref[N:NP] = jnp.zeros((NP - N, D), jnp.float32)


def _k1(x, W1, degp):
    return pl.pallas_call(
        _k1_body,
        out_shape=jax.ShapeDtypeStruct((NP, D), jnp.float32),
    )(x, W1, degp)


def _k2_body(p_ref, hs1_ref, degp_ref, b1_ref, w2_ref, o_ref):
    dinv = _dinv_from(degp_ref[...])
    tot = p_ref[0, :N] + p_ref[1, :N] + hs1_ref[0:N]
    h1 = jnp.maximum(tot * dinv + b1_ref[...], 0.0)
    h2 = jnp.dot(h1, w2_ref[...], preferred_element_type=jnp.float32)
    o_ref[0:N] = h2 * dinv
    o_ref[N:NP] = jnp.zeros((NP - N, D), jnp.float32)


def _k2(p, hs1, degp, b1, W2):
    return pl.pallas_call(
        _k2_body,
        out_shape=jax.ShapeDtypeStruct((NP, D), jnp.float32),
    )(p, hs1, degp, b1.reshape(1, D), W2)


def _k3_body(p_ref, hs2_ref, degp_ref, b2_ref, batch_ref, wih_ref, whh_ref,
             bih_ref, bhh_ref, o_ref):
    dinv = _dinv_from(degp_ref[...])
    h2 = (p_ref[0, :N] + p_ref[1, :N] + hs2_ref[0:N]) * dinv + b2_ref[...]

    seg = batch_ref[0:1, :]
    bids = lax.broadcasted_iota(jnp.int32, (B, N), 0)
    member = bids == seg

    q_star = jnp.zeros((B, 2 * D), jnp.float32)
    h = jnp.zeros((B, D), jnp.float32)
    c = jnp.zeros((B, D), jnp.float32)
    wih = wih_ref[...]
    whh = whh_ref[...]
    bias = bih_ref[...] + bhh_ref[...]

    for _ in range(STEPS):
        gates = (
            lax.dot_general(q_star, wih, (((1,), (1,)), ((), ())),
                            preferred_element_type=jnp.float32)
            + lax.dot_general(h, whh, (((1,), (1,)), ((), ())),
                              preferred_element_type=jnp.float32)
            + bias)
        ig = jax.nn.sigmoid(gates[:, 0:D])
        fg = jax.nn.sigmoid(gates[:, D:2 * D])
        gg = jnp.tanh(gates[:, 2 * D:3 * D])
        og = jax.nn.sigmoid(gates[:, 3 * D:4 * D])
        c = fg * c + ig * gg
        h = og * jnp.tanh(c)

        e = lax.dot_general(h, h2, (((1,), (1,)), ((), ())),
                            preferred_element_type=jnp.float32)
        em = jnp.where(member, e, -jnp.inf)
        emax = jnp.max(em, axis=1, keepdims=True)
        emax = jnp.where(emax < -3e38, 0.0, emax)
        a = jnp.exp(em - emax)
        ssum = jnp.sum(a, axis=1, keepdims=True)
        r = lax.dot_general(a, h2, (((1,), (0,)), ((), ())),
                            preferred_element_type=jnp.float32)
        r = r / (ssum + 1e-16)
        q_star = jnp.concatenate([h, r], axis=1)

    o_ref[...] = q_star


def _k3(p2, hs2, degp, b2, batch, w_ih, w_hh, b_ih, b_hh):
    batch8 = jnp.broadcast_to(batch[None, :], (8, N))
    return pl.pallas_call(
        _k3_body,
        out_shape=jax.ShapeDtypeStruct((B, 2 * D), jnp.float32),
    )(p2, hs2, degp, b2.reshape(1, D), batch8, w_ih, w_hh,
      b_ih.reshape(1, 4 * D), b_hh.reshape(1, 4 * D))


def kernel(x, edge_index, batch, W1, b1, W2, b2, w_ih, w_hh, b_ih, b_hh):
    pad = (N + jnp.arange(EPAD, dtype=edge_index.dtype) % (NP - N))
    src = jnp.concatenate([edge_index[0], pad]).reshape(EROWS, CHUNK)
    dst = jnp.concatenate([edge_index[1], pad]).reshape(EROWS, CHUNK)

    onehot = jnp.zeros((CHUNK, 16), jnp.float32).at[:, 0].set(1.0)
    zrow = jnp.zeros((RPS, 16), jnp.float32)
    zfeat = jnp.zeros((RPS, D), jnp.float32)

    degp = _deg_sc(dst, onehot, zrow)
    hs1 = _k1(x, W1, degp)
    p1 = _agg_sc(hs1, src, dst, zfeat)
    hs2 = _k2(p1, hs1, degp, b1, W2)
    p2 = _agg_sc(hs2, src, dst, zfeat)
    return _k3(p2, hs2, degp, b2, batch, w_ih, w_hh, b_ih, b_hh)
